# Initial kernel scaffold; baseline (speedup 1.0000x reference)
import jax
import jax.numpy as jnp
from jax.experimental import pallas as pl


def kernel(input_ids, W_word, W_pos):
    raise NotImplementedError("write your pallas kernel here")


# speedup vs baseline: 11.4283x; 11.4283x over previous
import functools

import jax
import jax.numpy as jnp
from jax import lax
from jax.experimental import pallas as pl
from jax.experimental.pallas import tpu as pltpu
from jax.experimental.pallas import tpu_sc as plsc

VOCAB = 100000
MAX_SEQ_LEN = 512
DIM = 128
B, L = 4096, 200

_INFO = plsc.get_sparse_core_info()
_NC, _NS = _INFO.num_cores, _INFO.num_subcores
_NW = _NC * _NS

_N_ROWS = B * L
_ROWS_PER_W = _N_ROWS // _NW
_CHUNK = 128
_N_CHUNKS = _ROWS_PER_W // _CHUNK
_NBUF = 4
_N_GROUPS = _N_CHUNKS // _NBUF


def _gather_kernel(table_hbm, ids_hbm, out_hbm, idx_v, rows_v, gsem):
    wid = lax.axis_index("s") * _NC + lax.axis_index("c")
    row0 = wid * _ROWS_PER_W
    chunk0 = wid * _N_CHUNKS

    pltpu.sync_copy(ids_hbm.at[pl.ds(chunk0, _N_CHUNKS)], idx_v)

    def start(j, b):
        pltpu.async_copy(table_hbm.at[idx_v.at[j]], rows_v.at[b], gsem)

    def wait(j, b):
        pltpu.make_async_copy(table_hbm.at[idx_v.at[j]], rows_v.at[b],
                              gsem).wait()

    def body(j, carry):
        start(j, 0)
        wait(j, 0)
        pltpu.sync_copy(rows_v.at[0],
                        out_hbm.at[pl.ds(row0 + j * _CHUNK, _CHUNK)])
        return carry

    lax.fori_loop(0, _N_CHUNKS, body, 0)


@functools.partial(jax.jit, static_argnames=())
def _token_gather(ids_2d, W_word):
    mesh = plsc.VectorSubcoreMesh(core_axis_name="c", subcore_axis_name="s")
    return pl.kernel(
        _gather_kernel,
        mesh=mesh,
        out_type=jax.ShapeDtypeStruct((_N_ROWS, DIM), jnp.float32),
        scratch_types=[
            pltpu.VMEM((_N_CHUNKS, _CHUNK), jnp.int32),
            pltpu.VMEM((_NBUF, _CHUNK, DIM), jnp.float32),
            pltpu.SemaphoreType.DMA,
        ],
    )(W_word, ids_2d)


_POS_BLK = 16


def _pos_kernel(pos_ref, out_ref):
    out_ref[...] = jnp.broadcast_to(pos_ref[...][None], out_ref.shape)


def _pos_broadcast(W_pos_l):
    return pl.pallas_call(
        _pos_kernel,
        grid=(B // _POS_BLK,),
        in_specs=[pl.BlockSpec((L, DIM), lambda i: (0, 0))],
        out_specs=pl.BlockSpec((_POS_BLK, L, DIM), lambda i: (i, 0, 0)),
        out_shape=jax.ShapeDtypeStruct((B, L, DIM), jnp.float32),
    )(W_pos_l)


def kernel(input_ids, W_word, W_pos):
    ids_2d = input_ids.astype(jnp.int32).reshape(_N_ROWS // _CHUNK, _CHUNK)
    X_token = _token_gather(ids_2d, W_word).reshape(B, L, DIM)
    X_pos = _pos_broadcast(W_pos[:L])
    return (X_token, X_pos)


# speedup vs baseline: 12.6431x; 1.1063x over previous
import functools

import jax
import jax.numpy as jnp
from jax import lax
from jax.experimental import pallas as pl
from jax.experimental.pallas import tpu as pltpu
from jax.experimental.pallas import tpu_sc as plsc

VOCAB = 100000
MAX_SEQ_LEN = 512
DIM = 128
B, L = 4096, 200

_INFO = plsc.get_sparse_core_info()
_NC, _NS = _INFO.num_cores, _INFO.num_subcores
_NW = _NC * _NS

_N_ROWS = B * L
_ROWS_PER_W = _N_ROWS // _NW
_CHUNK = 128
_N_CHUNKS = _ROWS_PER_W // _CHUNK
_NBUF = 4
_N_GROUPS = _N_CHUNKS // _NBUF


def _gather_kernel(table_hbm, ids_hbm, out_hbm, idx_v, rows_v, *gsems):
    wid = lax.axis_index("s") * _NC + lax.axis_index("c")
    row0 = wid * _ROWS_PER_W
    chunk0 = wid * _N_CHUNKS

    pltpu.sync_copy(ids_hbm.at[pl.ds(chunk0, _N_CHUNKS)], idx_v)

    def start(j, b):
        pltpu.async_copy(table_hbm.at[idx_v.at[j]], rows_v.at[b], gsems[b])

    def wait(j, b):
        pltpu.make_async_copy(table_hbm.at[idx_v.at[j]], rows_v.at[b],
                              gsems[b]).wait()

    for b in range(_NBUF):
        start(b, b)

    def body(g, carry):
        for b in range(_NBUF):
            j = g * _NBUF + b
            wait(j, b)
            pltpu.sync_copy(rows_v.at[b],
                            out_hbm.at[pl.ds(row0 + j * _CHUNK, _CHUNK)])
            jn = j + _NBUF

            @pl.when(jn < _N_CHUNKS)
            def _():
                start(jn, b)
        return carry

    lax.fori_loop(0, _N_GROUPS, body, 0)


@functools.partial(jax.jit, static_argnames=())
def _token_gather(ids_2d, W_word):
    mesh = plsc.VectorSubcoreMesh(core_axis_name="c", subcore_axis_name="s")
    return pl.kernel(
        _gather_kernel,
        mesh=mesh,
        out_type=jax.ShapeDtypeStruct((_N_ROWS, DIM), jnp.float32),
        scratch_types=[
            pltpu.VMEM((_N_CHUNKS, _CHUNK), jnp.int32),
            pltpu.VMEM((_NBUF, _CHUNK, DIM), jnp.float32),
        ] + [pltpu.SemaphoreType.DMA] * _NBUF,
    )(W_word, ids_2d)


_POS_BLK = 16


def _pos_kernel(pos_ref, out_ref):
    out_ref[...] = jnp.broadcast_to(pos_ref[...][None], out_ref.shape)


def _pos_broadcast(W_pos_l):
    return pl.pallas_call(
        _pos_kernel,
        grid=(B // _POS_BLK,),
        in_specs=[pl.BlockSpec((L, DIM), lambda i: (0, 0))],
        out_specs=pl.BlockSpec((_POS_BLK, L, DIM), lambda i: (i, 0, 0)),
        out_shape=jax.ShapeDtypeStruct((B, L, DIM), jnp.float32),
    )(W_pos_l)


def kernel(input_ids, W_word, W_pos):
    ids_2d = input_ids.astype(jnp.int32).reshape(_N_ROWS // _CHUNK, _CHUNK)
    X_token = _token_gather(ids_2d, W_word).reshape(B, L, DIM)
    X_pos = _pos_broadcast(W_pos[:L])
    return (X_token, X_pos)


# speedup vs baseline: 12.6480x; 1.0004x over previous
import functools

import jax
import jax.numpy as jnp
from jax import lax
from jax.experimental import pallas as pl
from jax.experimental.pallas import tpu as pltpu
from jax.experimental.pallas import tpu_sc as plsc

VOCAB = 100000
MAX_SEQ_LEN = 512
DIM = 128
B, L = 4096, 200

_INFO = plsc.get_sparse_core_info()
_NC, _NS = _INFO.num_cores, _INFO.num_subcores
_NW = _NC * _NS

_N_ROWS = B * L
_ROWS_PER_W = _N_ROWS // _NW
_CHUNK = 128
_N_CHUNKS = _ROWS_PER_W // _CHUNK
_NBUF = 5
_N_GROUPS = _N_CHUNKS // _NBUF


def _gather_kernel(table_hbm, ids_hbm, out_hbm, idx_v, rows_v, *gsems):
    wid = lax.axis_index("s") * _NC + lax.axis_index("c")
    row0 = wid * _ROWS_PER_W
    chunk0 = wid * _N_CHUNKS

    pltpu.sync_copy(ids_hbm.at[pl.ds(chunk0, _N_CHUNKS)], idx_v)

    gin = gsems[:_NBUF]
    gout = gsems[_NBUF:]

    def start(j, b):
        pltpu.async_copy(table_hbm.at[idx_v.at[j]], rows_v.at[b], gin[b])

    def wait(j, b):
        pltpu.make_async_copy(table_hbm.at[idx_v.at[j]], rows_v.at[b],
                              gin[b]).wait()

    def start_out(j, b):
        pltpu.async_copy(rows_v.at[b],
                         out_hbm.at[pl.ds(row0 + j * _CHUNK, _CHUNK)],
                         gout[b])

    def wait_out(j, b):
        pltpu.make_async_copy(rows_v.at[b],
                              out_hbm.at[pl.ds(row0 + j * _CHUNK, _CHUNK)],
                              gout[b]).wait()

    for b in range(_NBUF):
        start(b, b)

    def body(g, carry):
        for b in range(_NBUF):
            j = g * _NBUF + b
            wait(j, b)
            start_out(j, b)
        for b in range(_NBUF):
            j = g * _NBUF + b
            jn = j + _NBUF
            wait_out(j, b)

            @pl.when(jn < _N_CHUNKS)
            def _():
                start(jn, b)
        return carry

    lax.fori_loop(0, _N_GROUPS, body, 0)


@functools.partial(jax.jit, static_argnames=())
def _token_gather(ids_2d, W_word):
    mesh = plsc.VectorSubcoreMesh(core_axis_name="c", subcore_axis_name="s")
    return pl.kernel(
        _gather_kernel,
        mesh=mesh,
        out_type=jax.ShapeDtypeStruct((_N_ROWS, DIM), jnp.float32),
        scratch_types=[
            pltpu.VMEM((_N_CHUNKS, _CHUNK), jnp.int32),
            pltpu.VMEM((_NBUF, _CHUNK, DIM), jnp.float32),
        ] + [pltpu.SemaphoreType.DMA] * (2 * _NBUF),
    )(W_word, ids_2d)


_POS_BLK = 16


def _pos_kernel(pos_ref, out_ref):
    out_ref[...] = jnp.broadcast_to(pos_ref[...][None], out_ref.shape)


def _pos_broadcast(W_pos_l):
    return pl.pallas_call(
        _pos_kernel,
        grid=(B // _POS_BLK,),
        in_specs=[pl.BlockSpec((L, DIM), lambda i: (0, 0))],
        out_specs=pl.BlockSpec((_POS_BLK, L, DIM), lambda i: (i, 0, 0)),
        out_shape=jax.ShapeDtypeStruct((B, L, DIM), jnp.float32),
    )(W_pos_l)


def kernel(input_ids, W_word, W_pos):
    ids_2d = input_ids.astype(jnp.int32).reshape(_N_ROWS // _CHUNK, _CHUNK)
    X_token = _token_gather(ids_2d, W_word).reshape(B, L, DIM)
    X_pos = _pos_broadcast(W_pos[:L])
    return (X_token, X_pos)
